# Initial kernel scaffold; baseline (speedup 1.0000x reference)
import jax
import jax.numpy as jnp
from jax.experimental import pallas as pl


def kernel(x, params, edge_index, batch, target):
    raise NotImplementedError("write your pallas kernel here")


# speedup vs baseline: 7.5171x; 7.5171x over previous
import functools

import jax
import jax.numpy as jnp
from jax import lax
from jax.experimental import pallas as pl
from jax.experimental.pallas import tpu as pltpu
from jax.experimental.pallas import tpu_sc as plsc

N_NODES = 10000
N_EDGES = 320000
N_GRAPHS = 32
EPS = 1e-7

NC, NS = 2, 16
NW = NC * NS
EPW = N_EDGES // NW
KB = 40
SB = 5
GRP = KB * SB
NG = EPW // GRP
NPAD = 10112
RPW = NPAD // NS


def _edge_agg_body(tp_hbm, src_hbm, dst_hbm, zz_hbm, out_hbm,
                   acc, sidx, didx, rows, gsem, ssem):
    c = lax.axis_index("c")
    s = lax.axis_index("s")
    wid = c * NS + s

    pltpu.sync_copy(zz_hbm.at[pl.ds(s * RPW, RPW)],
                    acc.at[pl.ds(s * RPW, RPW)])
    plsc.subcore_barrier()

    def group(gi, carry):
        pltpu.sync_copy(src_hbm.at[wid].at[gi], sidx)
        pltpu.sync_copy(dst_hbm.at[wid].at[gi], didx)
        gdescs = []
        for j in range(SB):
            gdescs.append(pltpu.async_copy(
                tp_hbm.at[sidx.at[j]],
                rows.at[pl.ds(j * KB, KB)], gsem))
        for d_ in gdescs:
            d_.wait()
        sdescs = []
        for j in range(SB):
            sdescs.append(pltpu.async_copy(
                rows.at[pl.ds(j * KB, KB)],
                acc.at[didx.at[j]], ssem, add=True))
        for d_ in sdescs:
            d_.wait()
        return carry

    lax.fori_loop(0, NG, group, 0)
    plsc.subcore_barrier()
    pltpu.sync_copy(acc.at[pl.ds(s * RPW, RPW)],
                    out_hbm.at[c].at[pl.ds(s * RPW, RPW)])


def _edge_agg(tp, src3, dst3, zz):
    mesh = plsc.VectorSubcoreMesh(core_axis_name="c", subcore_axis_name="s")
    kern = pl.kernel(
        _edge_agg_body,
        out_type=jax.ShapeDtypeStruct((NC, NPAD, 128), jnp.float32),
        mesh=mesh,
        scratch_types=[
            pltpu.VMEM_SHARED((NPAD, 128), jnp.float32),
            pltpu.VMEM((SB, KB), jnp.int32),
            pltpu.VMEM((SB, KB), jnp.int32),
            pltpu.VMEM((GRP, 128), jnp.float32),
            pltpu.SemaphoreType.DMA,
            pltpu.SemaphoreType.DMA,
        ],
    )
    return kern(tp, src3, dst3, zz)


def _tp_chunks(t, p_, n_chunks, tp_ref):
    for cc in range(n_chunks):
        lo = cc * 64
        tp_ref[cc, :, :] = jnp.concatenate(
            [t[:, lo:lo + 64], p_[:, lo:lo + 64]], axis=1)


def _prep0_body(x_ref, tp_ref):
    r = jnp.maximum(x_ref[...], 0.0) + EPS
    t = jnp.exp(r)
    p_ = r * t
    tp_ref[0, :, :] = jnp.concatenate([t[:, :64], p_[:, :64]], axis=1)
    tp_ref[1, :, :] = jnp.concatenate([t[:, 64:], p_[:, 64:]], axis=1)


def _prep0(x):
    blk = 1000
    return pl.pallas_call(
        _prep0_body,
        grid=(N_NODES // blk,),
        in_specs=[pl.BlockSpec((blk, 128), lambda i: (i, 0))],
        out_specs=pl.BlockSpec((2, blk, 128), lambda i: (0, i, 0)),
        out_shape=jax.ShapeDtypeStruct((2, N_NODES, 128), jnp.float32),
    )(x)


def _post_body(n_in, n_out, h_ref, w1_ref, b1_ref, w2_ref, b2_ref,
               *refs):
    part_refs = refs[:n_in]
    if n_out:
        h_out_ref = refs[n_in]
        tp_refs = refs[n_in + 1:]
    else:
        h_out_ref = refs[n_in]
        tp_refs = ()
    aggs = []
    for cc in range(n_in):
        sa = part_refs[cc][0, :, :] + part_refs[cc][1, :, :]
        aggs.append(sa[:, 64:] / (sa[:, :64] + 1e-30))
    agg = jnp.concatenate(aggs, axis=1) if n_in > 1 else aggs[0]
    o = agg + h_ref[...]
    z = jnp.maximum(
        jnp.dot(o, w1_ref[...], preferred_element_type=jnp.float32)
        + b1_ref[...], 0.0)
    h2 = jnp.maximum(
        jnp.dot(z, w2_ref[...], preferred_element_type=jnp.float32)
        + b2_ref[...], 0.0)
    h_out_ref[...] = h2
    if n_out:
        r = h2 + EPS
        t = jnp.exp(r)
        p_ = r * t
        _tp_chunks(t, p_, n_out, tp_refs[0])


def _post(h, parts, pc, blk, n_out):
    n_in = len(parts)
    f_in = 64 * n_in
    hdim = pc['W1'].shape[1]
    f2 = pc['W2'].shape[1]
    out_shapes = [jax.ShapeDtypeStruct((N_NODES, f2), jnp.float32)]
    out_specs = [pl.BlockSpec((blk, f2), lambda i: (i, 0))]
    if n_out:
        out_shapes.append(
            jax.ShapeDtypeStruct((n_out, N_NODES, 128), jnp.float32))
        out_specs.append(
            pl.BlockSpec((n_out, blk, 128), lambda i: (0, i, 0)))
    res = pl.pallas_call(
        functools.partial(_post_body, n_in, n_out),
        grid=(N_NODES // blk,),
        in_specs=[
            pl.BlockSpec((blk, f_in), lambda i: (i, 0)),
            pl.BlockSpec(pc['W1'].shape, lambda i: (0, 0)),
            pl.BlockSpec((1, hdim), lambda i: (0, 0)),
            pl.BlockSpec(pc['W2'].shape, lambda i: (0, 0)),
            pl.BlockSpec((1, f2), lambda i: (0, 0)),
        ] + [pl.BlockSpec((2, blk, 128), lambda i: (0, i, 0))
             for _ in range(n_in)],
        out_specs=out_specs,
        out_shape=out_shapes,
    )(h, pc['W1'], pc['b1'].reshape(1, -1), pc['W2'],
      pc['b2'].reshape(1, -1), *parts)
    return res if n_out else (res[0], None)


def _pool_head_body(ngrid, blk, batch_ref, h_ref,
                    w1_ref, b1_ref, w2_ref, b2_ref, w3_ref, b3_ref,
                    out_ref, sums, cnts):
    i = pl.program_id(0)

    @pl.when(i == 0)
    def _init():
        sums[...] = jnp.zeros_like(sums)
        cnts[...] = jnp.zeros_like(cnts)

    b = batch_ref[...]
    oh = (b == lax.broadcasted_iota(jnp.int32, (blk, N_GRAPHS), 1)
          ).astype(jnp.float32)
    hv = h_ref[...]
    sums[...] += lax.dot_general(oh, hv, (((0,), (0,)), ((), ())),
                                 preferred_element_type=jnp.float32)
    cnts[...] += lax.dot_general(
        oh, jnp.ones((blk, 128), jnp.float32), (((0,), (0,)), ((), ())),
        preferred_element_type=jnp.float32)

    @pl.when(i == ngrid - 1)
    def _fin():
        g = sums[...] / jnp.maximum(cnts[...][:, 0:1], 1.0)
        g = jnp.maximum(jnp.dot(g, w1_ref[...],
                                preferred_element_type=jnp.float32)
                        + b1_ref[...], 0.0)
        g = jnp.maximum(jnp.dot(g, w2_ref[...],
                                preferred_element_type=jnp.float32)
                        + b2_ref[...], 0.0)
        g = jnp.maximum(jnp.dot(g, w3_ref[...],
                                preferred_element_type=jnp.float32)
                        + b3_ref[...], 0.0)
        out_ref[...] = g


def _pool_head(batch2, h3, p):
    blk = 1000
    ngrid = N_NODES // blk
    return pl.pallas_call(
        functools.partial(_pool_head_body, ngrid, blk),
        grid=(ngrid,),
        in_specs=[
            pl.BlockSpec((blk, 1), lambda i: (i, 0)),
            pl.BlockSpec((blk, 1024), lambda i: (i, 0)),
            pl.BlockSpec((1024, 512), lambda i: (0, 0)),
            pl.BlockSpec((1, 512), lambda i: (0, 0)),
            pl.BlockSpec((512, 1024), lambda i: (0, 0)),
            pl.BlockSpec((1, 1024), lambda i: (0, 0)),
            pl.BlockSpec((1024, 128), lambda i: (0, 0)),
            pl.BlockSpec((1, 128), lambda i: (0, 0)),
        ],
        out_specs=pl.BlockSpec((N_GRAPHS, 128), lambda i: (0, 0)),
        out_shape=jax.ShapeDtypeStruct((N_GRAPHS, 128), jnp.float32),
        scratch_shapes=[
            pltpu.VMEM((N_GRAPHS, 1024), jnp.float32),
            pltpu.VMEM((N_GRAPHS, 128), jnp.float32),
        ],
    )(batch2, h3, p['fc_g1_W'], p['fc_g1_b'].reshape(1, -1),
      p['fc_g2_W'], p['fc_g2_b'].reshape(1, -1),
      p['fc_g3_W'], p['fc_g3_b'].reshape(1, -1))


def _protein_body(t_ref, embt_ref, w8t_ref, cb_ref, out_ref):
    b = pl.program_id(0)
    t = t_ref[pl.ds(b, 1), :]
    oht = (t == lax.broadcasted_iota(jnp.int32, (26, 1000), 0)
           ).astype(jnp.float32)
    embmt = jnp.dot(embt_ref[...], oht,
                    preferred_element_type=jnp.float32)
    ip = lax.broadcasted_iota(jnp.int32, (128, 128), 0)
    iq = lax.broadcasted_iota(jnp.int32, (128, 128), 1)
    valid = (ip < 121)
    convt = jnp.zeros((128, 32), jnp.float32)
    for k in range(8):
        ckt = jnp.dot(embmt, w8t_ref[k],
                      preferred_element_type=jnp.float32)
        mkt = ((iq - ip == k) & valid).astype(jnp.float32)
        convt = convt + jnp.dot(mkt, ckt,
                                preferred_element_type=jnp.float32)
    pmask = (lax.broadcasted_iota(jnp.int32, (128, 32), 0) < 121
             ).astype(jnp.float32)
    convt = convt + cb_ref[...] * pmask
    out_ref[0, :, :] = convt


def _protein(target, embt, w8t, convb):
    return pl.pallas_call(
        _protein_body,
        grid=(N_GRAPHS,),
        in_specs=[
            pl.BlockSpec((N_GRAPHS, 1000), lambda b: (0, 0)),
            pl.BlockSpec((128, 26), lambda b: (0, 0)),
            pl.BlockSpec((8, 1000, 32), lambda b: (0, 0, 0)),
            pl.BlockSpec((1, 32), lambda b: (0, 0)),
        ],
        out_specs=pl.BlockSpec((1, 128, 32), lambda b: (b, 0, 0)),
        out_shape=jax.ShapeDtypeStruct((N_GRAPHS, 128, 32), jnp.float32),
    )(target, embt, w8t, convb.reshape(1, 32))


def _head_body(g3_ref, cf_ref, wxt_ref, bxt_ref, f1_ref, fb1_ref,
               f2_ref, fb2_ref, wo_ref, bo_ref, out_ref):
    xt = jnp.dot(cf_ref[...], wxt_ref[...],
                 preferred_element_type=jnp.float32) + bxt_ref[...]
    f1 = f1_ref[...]
    y = jnp.maximum(
        jnp.dot(g3_ref[...], f1[:128, :],
                preferred_element_type=jnp.float32)
        + jnp.dot(xt, f1[128:, :], preferred_element_type=jnp.float32)
        + fb1_ref[...], 0.0)
    y = jnp.maximum(jnp.dot(y, f2_ref[...],
                            preferred_element_type=jnp.float32)
                    + fb2_ref[...], 0.0)
    out_ref[...] = (jnp.dot(y, wo_ref[...],
                            preferred_element_type=jnp.float32)
                    + bo_ref[...])


def _head(g3, convflat, wxtpad, p):
    return pl.pallas_call(
        _head_body,
        grid=(1,),
        in_specs=[
            pl.BlockSpec((N_GRAPHS, 128), lambda i: (0, 0)),
            pl.BlockSpec((N_GRAPHS, 4096), lambda i: (0, 0)),
            pl.BlockSpec((4096, 128), lambda i: (0, 0)),
            pl.BlockSpec((1, 128), lambda i: (0, 0)),
            pl.BlockSpec((256, 1024), lambda i: (0, 0)),
            pl.BlockSpec((1, 1024), lambda i: (0, 0)),
            pl.BlockSpec((1024, 512), lambda i: (0, 0)),
            pl.BlockSpec((1, 512), lambda i: (0, 0)),
            pl.BlockSpec((512, 1), lambda i: (0, 0)),
            pl.BlockSpec((1, 1), lambda i: (0, 0)),
        ],
        out_specs=pl.BlockSpec((N_GRAPHS, 1), lambda i: (0, 0)),
        out_shape=jax.ShapeDtypeStruct((N_GRAPHS, 1), jnp.float32),
    )(g3, convflat, wxtpad, p['fc1_xt_b'].reshape(1, -1),
      p['fc1_W'], p['fc1_b'].reshape(1, -1),
      p['fc2_W'], p['fc2_b'].reshape(1, -1),
      p['out_W'], p['out_b'].reshape(1, -1))


def kernel(x, params, edge_index, batch, target):
    p = params
    src3 = edge_index[0].reshape(NW, NG, SB, KB)
    dst3 = edge_index[1].reshape(NW, NG, SB, KB)
    zz = jnp.zeros((NPAD, 128), jnp.float32)

    tp0 = _prep0(x)
    parts1 = [_edge_agg(tp0[c], src3, dst3, zz) for c in range(2)]
    h1, tp1 = _post(x, parts1, p['c1'], blk=1000, n_out=2)
    parts2 = [_edge_agg(tp1[c], src3, dst3, zz) for c in range(2)]
    h2, tp2 = _post(h1, parts2, p['c2'], blk=1000, n_out=8)
    parts3 = [_edge_agg(tp2[c], src3, dst3, zz) for c in range(8)]
    h3, _ = _post(h2, parts3, p['c3'], blk=400, n_out=0)

    g3 = _pool_head(batch.reshape(N_NODES, 1), h3, p)
    w8t = jnp.transpose(p['convW'], (2, 1, 0))
    convpad = _protein(target, jnp.transpose(p['emb']), w8t, p['convb'])
    convflat = jnp.transpose(convpad, (0, 2, 1)).reshape(N_GRAPHS, 4096)
    wxtpad = jnp.pad(p['fc1_xt_W'].reshape(32, 121, 128),
                     ((0, 0), (0, 7), (0, 0))).reshape(4096, 128)
    return _head(g3, convflat, wxtpad, p)
